# Initial kernel scaffold; baseline (speedup 1.0000x reference)
import jax
import jax.numpy as jnp
from jax.experimental import pallas as pl


def kernel(x, edge_index, Wl, Wr, bc, gamma, beta, alpha, W1, b1, W2, b2, W3, b3):
    raise NotImplementedError("write your pallas kernel here")


# speedup vs baseline: 7.6224x; 7.6224x over previous
import jax
import jax.numpy as jnp
from jax import lax
from jax.experimental import pallas as pl
from jax.experimental.pallas import tpu as pltpu
from jax.experimental.pallas import tpu_sc as plsc

_N = 10000
_D = 128
_E = 320000
_NC = 2
_NS = 16
_NW = _NC * _NS
_EPW = _E // _NW
_C = 125
_NCH = _EPW // _C
_NP = 10240
_RPT = _NP // _NS

_mesh = plsc.VectorSubcoreMesh(core_axis_name="c", subcore_axis_name="s")


def _agg_body(h_hbm, src_hbm, dst_hbm, zeros_hbm, out_hbm,
              src_v, dst_v, rows_v, agg_sh, gsem):
    cid = lax.axis_index("c")
    sid = lax.axis_index("s")
    wid = sid * _NC + cid
    pltpu.sync_copy(zeros_hbm.at[pl.ds(sid * _RPT, _RPT)],
                    agg_sh.at[pl.ds(sid * _RPT, _RPT)])
    pltpu.sync_copy(src_hbm.at[wid], src_v)
    pltpu.sync_copy(dst_hbm.at[wid], dst_v)
    plsc.subcore_barrier()

    def chunk(j, carry):
        pltpu.async_copy(h_hbm.at[src_v.at[j]], rows_v, gsem).wait()
        pltpu.sync_copy(rows_v, agg_sh.at[dst_v.at[j]], add=True)
        return carry

    lax.fori_loop(0, _NCH, chunk, 0)
    plsc.subcore_barrier()
    pltpu.sync_copy(agg_sh.at[pl.ds(sid * _RPT, _RPT)],
                    out_hbm.at[cid, pl.ds(sid * _RPT, _RPT)])


_agg = pl.kernel(
    _agg_body,
    out_type=jax.ShapeDtypeStruct((_NC, _NP, _D), jnp.float32),
    mesh=_mesh,
    scratch_types=[
        pltpu.VMEM((_NCH, _C), jnp.int32),
        pltpu.VMEM((_NCH, _C), jnp.int32),
        pltpu.VMEM((_C, _D), jnp.float32),
        pltpu.VMEM_SHARED((_NP, _D), jnp.float32),
        pltpu.SemaphoreType.DMA,
    ],
)


def _deg_body(dst_hbm, ones_hbm, zeros_hbm, out_hbm,
              dst_v, ones_v, deg_sh):
    cid = lax.axis_index("c")
    sid = lax.axis_index("s")
    wid = sid * _NC + cid
    pltpu.sync_copy(zeros_hbm.at[pl.ds(sid * _RPT, _RPT)],
                    deg_sh.at[pl.ds(sid * _RPT, _RPT)])
    pltpu.sync_copy(dst_hbm.at[wid], dst_v)
    pltpu.sync_copy(ones_hbm, ones_v)
    plsc.subcore_barrier()

    def chunk(j, carry):
        pltpu.sync_copy(ones_v, deg_sh.at[dst_v.at[j]], add=True)
        return carry

    lax.fori_loop(0, _NCH, chunk, 0)
    plsc.subcore_barrier()
    pltpu.sync_copy(deg_sh.at[pl.ds(sid * _RPT, _RPT)],
                    out_hbm.at[cid, pl.ds(sid * _RPT, _RPT)])


_deg = pl.kernel(
    _deg_body,
    out_type=jax.ShapeDtypeStruct((_NC, _NP, _D), jnp.float32),
    mesh=_mesh,
    scratch_types=[
        pltpu.VMEM((_NCH, _C), jnp.int32),
        pltpu.VMEM((_C, _D), jnp.float32),
        pltpu.VMEM_SHARED((_NP, _D), jnp.float32),
    ],
)


def _dense_body(p_ref, degp_ref, h_ref, wl_ref, wr_ref, bvec_ref, o_ref):
    agg = p_ref[0, :_N] + p_ref[1, :_N]
    deg = degp_ref[0, :_N, 0:1] + degp_ref[1, :_N, 0:1]
    mean = agg / jnp.maximum(deg, 1.0)
    h = h_ref[...]
    t = (jnp.dot(mean, wl_ref[...], preferred_element_type=jnp.float32)
         + jnp.dot(h, wr_ref[...], preferred_element_type=jnp.float32)
         + bvec_ref[0:1, :])
    gamma_v = bvec_ref[1:2, :]
    beta_v = bvec_ref[2:3, :]
    alpha_v = bvec_ref[3:4, :]
    mu = jnp.mean(t, axis=0, keepdims=True)
    o = t - alpha_v * mu
    var = jnp.mean(o * o, axis=0, keepdims=True)
    hn = gamma_v * o / jnp.sqrt(var + 1e-5) + beta_v
    o_ref[...] = jnp.maximum(hn, 0.0)


_dense = pl.pallas_call(
    _dense_body,
    out_shape=jax.ShapeDtypeStruct((_N, _D), jnp.float32),
)


def _heads_body(h_ref, w1_ref, b1_ref, w2_ref, b2_ref,
                w3a, w3b, w3c, w3d, w3e, b3_ref,
                o0, o1, o2, o3, o4):
    h = h_ref[...]
    w3s = (w3a, w3b, w3c, w3d, w3e)
    outs = (o0, o1, o2, o3, o4)
    for i in range(5):
        z = jnp.maximum(
            jnp.dot(h, w1_ref[i], preferred_element_type=jnp.float32)
            + b1_ref[i:i + 1, :], 0.0)
        z = jnp.maximum(
            jnp.dot(z, w2_ref[i], preferred_element_type=jnp.float32)
            + b2_ref[i:i + 1, :], 0.0)
        o = w3s[i][...]
        ncols = o.shape[1]
        outs[i][...] = (jnp.dot(z, o, preferred_element_type=jnp.float32)
                        + b3_ref[i:i + 1, :ncols])


def kernel(x, edge_index, Wl, Wr, bc, gamma, beta, alpha, W1, b1, W2, b2, W3, b3):
    src = edge_index[0].reshape(_NW, _NCH, _C)
    dst = edge_index[1].reshape(_NW, _NCH, _C)
    zeros = jnp.zeros((_NP, _D), jnp.float32)
    ones = jnp.ones((_C, _D), jnp.float32)

    degp = _deg(dst, ones, zeros)

    h = x
    for i in range(5):
        parts = _agg(h, src, dst, zeros)
        bvec = jnp.stack([bc[i], gamma[i], beta[i], alpha[i]])
        h = _dense(parts, degp, h, Wl[i], Wr[i], bvec)

    outs_dims = tuple(w.shape[1] for w in W3)
    b3_pad = jnp.stack([jnp.pad(b, (0, 8 - b.shape[0])) for b in b3])
    heads = pl.pallas_call(
        _heads_body,
        out_shape=tuple(jax.ShapeDtypeStruct((_N, o), jnp.float32)
                        for o in outs_dims),
    )
    return heads(h, W1, b1, W2, b2, *W3, b3_pad)
